# baseline (speedup 1.0000x reference)
import jax
import jax.numpy as jnp
from jax.experimental import pallas as pl
from jax.experimental.pallas import tpu as pltpu

_MARGIN = 0.0
_MOMENTUM = 0.95
_MAX_S = 20.0
_PREV_S = 20.0
_RUNNING_B = 1000.0
_RUNNING_COS = 0.7

_BR = 8
_LANES = 128


def _pass1_kernel(y_sm, x_ref, *args):
    seg_refs = args[:_BR]
    sums_ref, tgt_ref = args[_BR], args[_BR + 1]
    i = pl.program_id(0)
    x = x_ref[...]
    sums_ref[...] = jnp.sum(jnp.exp(x * _PREV_S), axis=1).reshape(1, 1, _BR)
    parts = []
    for r in range(_BR):
        lane = y_sm[i * _BR + r] % _LANES
        m = jax.lax.broadcasted_iota(jnp.int32, (1, _LANES), 1) == lane
        parts.append(jnp.where(m, seg_refs[r][pl.ds(r, 1), :], 0.0))
    tgt = jnp.sum(jnp.concatenate(parts, axis=0), axis=1)
    tgt_ref[...] = tgt.reshape(1, 1, _BR)


def _pass2_kernel(s_ref, x_ref, sums_ref):
    x = x_ref[...]
    sums_ref[...] = jnp.sum(jnp.exp(x * s_ref[0]), axis=1).reshape(1, 1, _BR)


def kernel(cosine, y_true):
    B, C = cosine.shape
    y_true = y_true.astype(jnp.int32)
    nb = B // _BR

    def _seg_spec(r):
        return pl.BlockSpec(
            (_BR, _LANES), lambda i, y: (i, y[i * _BR + r] // _LANES)
        )

    grid_spec = pltpu.PrefetchScalarGridSpec(
        num_scalar_prefetch=1,
        grid=(nb,),
        in_specs=[
            pl.BlockSpec((_BR, C), lambda i, y: (i, 0)),
            *[_seg_spec(r) for r in range(_BR)],
        ],
        out_specs=[
            pl.BlockSpec((1, 1, _BR), lambda i, y: (i, 0, 0)),
            pl.BlockSpec((1, 1, _BR), lambda i, y: (i, 0, 0)),
        ],
    )
    sums3, tgt3 = pl.pallas_call(
        _pass1_kernel,
        grid_spec=grid_spec,
        out_shape=[
            jax.ShapeDtypeStruct((nb, 1, _BR), jnp.float32),
            jax.ShapeDtypeStruct((nb, 1, _BR), jnp.float32),
        ],
        compiler_params=pltpu.CompilerParams(
            dimension_semantics=("arbitrary",)
        ),
    )(y_true, cosine, *([cosine] * _BR))
    sums = sums3.reshape(B)
    tgt = tgt3.reshape(B)

    total = jnp.sum(sums)
    b_batch = (total - jnp.sum(jnp.exp(tgt * _PREV_S))) / B
    med_cos = jnp.median(tgt)
    running_b = _RUNNING_B * _MOMENTUM + b_batch * (1.0 - _MOMENTUM)
    running_cos = _RUNNING_COS * _MOMENTUM + med_cos * (1.0 - _MOMENTUM)
    prev_s = jnp.log(running_b) / (jnp.maximum(running_cos, 0.7) - _MARGIN)
    prev_s = jnp.minimum(prev_s, _MAX_S)

    def _fast(_):
        return jnp.mean(jnp.log(sums) - prev_s * tgt)

    def _slow(_):
        sums2 = pl.pallas_call(
            _pass2_kernel,
            grid=(nb,),
            in_specs=[
                pl.BlockSpec(memory_space=pltpu.SMEM),
                pl.BlockSpec((_BR, C), lambda i: (i, 0)),
            ],
            out_specs=pl.BlockSpec((1, 1, _BR), lambda i: (i, 0, 0)),
            out_shape=jax.ShapeDtypeStruct((nb, 1, _BR), jnp.float32),
        )(prev_s[None], cosine)
        return jnp.mean(jnp.log(sums2.reshape(B)) - prev_s * tgt)

    return jax.lax.cond(prev_s == _PREV_S, _fast, _slow, None)


# speedup vs baseline: 1.0012x; 1.0012x over previous
import jax
import jax.numpy as jnp
from jax.experimental import pallas as pl
from jax.experimental.pallas import tpu as pltpu

_MARGIN = 0.0
_MOMENTUM = 0.95
_MAX_S = 20.0
_PREV_S = 20.0
_RUNNING_B = 1000.0
_RUNNING_COS = 0.7

_BR = 8
_LANES = 128


def _pass1_kernel(y_sm, x_ref, *args):
    seg_refs = args[:_BR]
    sums_ref, tgt_ref = args[_BR], args[_BR + 1]
    i = pl.program_id(0)
    x = x_ref[...]
    sums_ref[...] = jnp.sum(jnp.exp(x * _PREV_S), axis=1).reshape(1, 1, _BR)
    parts = []
    for r in range(_BR):
        lane = y_sm[i * _BR + r] % _LANES
        m = jax.lax.broadcasted_iota(jnp.int32, (1, _LANES), 1) == lane
        parts.append(jnp.where(m, seg_refs[r][pl.ds(r, 1), :], 0.0))
    tgt = jnp.sum(jnp.concatenate(parts, axis=0), axis=1)
    tgt_ref[...] = tgt.reshape(1, 1, _BR)


def _pass2_kernel(s_ref, x_ref, sums_ref):
    x = x_ref[...]
    sums_ref[...] = jnp.sum(jnp.exp(x * s_ref[0]), axis=1).reshape(1, 1, _BR)


def kernel(cosine, y_true):
    B, C = cosine.shape
    y_true = y_true.astype(jnp.int32)
    nb = B // _BR

    def _seg_spec(r):
        return pl.BlockSpec(
            (_BR, _LANES), lambda i, y: (i, y[i * _BR + r] // _LANES)
        )

    grid_spec = pltpu.PrefetchScalarGridSpec(
        num_scalar_prefetch=1,
        grid=(nb,),
        in_specs=[
            pl.BlockSpec((_BR, C), lambda i, y: (i, 0)),
            *[_seg_spec(r) for r in range(_BR)],
        ],
        out_specs=[
            pl.BlockSpec((1, 1, _BR), lambda i, y: (i, 0, 0)),
            pl.BlockSpec((1, 1, _BR), lambda i, y: (i, 0, 0)),
        ],
    )
    sums3, tgt3 = pl.pallas_call(
        _pass1_kernel,
        grid_spec=grid_spec,
        out_shape=[
            jax.ShapeDtypeStruct((nb, 1, _BR), jnp.float32),
            jax.ShapeDtypeStruct((nb, 1, _BR), jnp.float32),
        ],
        compiler_params=pltpu.CompilerParams(
            dimension_semantics=("parallel",)
        ),
    )(y_true, cosine, *([cosine] * _BR))
    sums = sums3.reshape(B)
    tgt = tgt3.reshape(B)

    total = jnp.sum(sums)
    b_batch = (total - jnp.sum(jnp.exp(tgt * _PREV_S))) / B
    med_cos = jnp.median(tgt)
    running_b = _RUNNING_B * _MOMENTUM + b_batch * (1.0 - _MOMENTUM)
    running_cos = _RUNNING_COS * _MOMENTUM + med_cos * (1.0 - _MOMENTUM)
    prev_s = jnp.log(running_b) / (jnp.maximum(running_cos, 0.7) - _MARGIN)
    prev_s = jnp.minimum(prev_s, _MAX_S)

    def _fast(_):
        return jnp.mean(jnp.log(sums) - prev_s * tgt)

    def _slow(_):
        sums2 = pl.pallas_call(
            _pass2_kernel,
            grid=(nb,),
            in_specs=[
                pl.BlockSpec(memory_space=pltpu.SMEM),
                pl.BlockSpec((_BR, C), lambda i: (i, 0)),
            ],
            out_specs=pl.BlockSpec((1, 1, _BR), lambda i: (i, 0, 0)),
            out_shape=jax.ShapeDtypeStruct((nb, 1, _BR), jnp.float32),
        )(prev_s[None], cosine)
        return jnp.mean(jnp.log(sums2.reshape(B)) - prev_s * tgt)

    return jax.lax.cond(prev_s == _PREV_S, _fast, _slow, None)


# speedup vs baseline: 1.0820x; 1.0808x over previous
import jax
import jax.numpy as jnp
from jax.experimental import pallas as pl
from jax.experimental.pallas import tpu as pltpu

_MARGIN = 0.0
_MOMENTUM = 0.95
_MAX_S = 20.0
_PREV_S = 20.0
_RUNNING_B = 1000.0
_RUNNING_COS = 0.7

_BR = 16
_LANES = 128


def _pass1_kernel(y_sm, x_ref, *args):
    seg_refs = args[:_BR]
    sums_ref, tgt_ref = args[_BR], args[_BR + 1]
    i = pl.program_id(0)
    x = x_ref[...]
    sums_ref[...] = jnp.sum(jnp.exp(x * _PREV_S), axis=1).reshape(1, 1, _BR)
    parts = []
    for r in range(_BR):
        lane = y_sm[i * _BR + r] % _LANES
        m = jax.lax.broadcasted_iota(jnp.int32, (1, _LANES), 1) == lane
        parts.append(jnp.where(m, seg_refs[r][pl.ds(r, 1), :], 0.0))
    tgt = jnp.sum(jnp.concatenate(parts, axis=0), axis=1)
    tgt_ref[...] = tgt.reshape(1, 1, _BR)


def _pass2_kernel(s_ref, x_ref, sums_ref):
    x = x_ref[...]
    sums_ref[...] = jnp.sum(jnp.exp(x * s_ref[0]), axis=1).reshape(1, 1, _BR)


def kernel(cosine, y_true):
    B, C = cosine.shape
    y_true = y_true.astype(jnp.int32)
    nb = B // _BR

    def _seg_spec(r):
        return pl.BlockSpec(
            (_BR, _LANES), lambda i, y: (i, y[i * _BR + r] // _LANES)
        )

    grid_spec = pltpu.PrefetchScalarGridSpec(
        num_scalar_prefetch=1,
        grid=(nb,),
        in_specs=[
            pl.BlockSpec((_BR, C), lambda i, y: (i, 0)),
            *[_seg_spec(r) for r in range(_BR)],
        ],
        out_specs=[
            pl.BlockSpec((1, 1, _BR), lambda i, y: (i, 0, 0)),
            pl.BlockSpec((1, 1, _BR), lambda i, y: (i, 0, 0)),
        ],
    )
    sums3, tgt3 = pl.pallas_call(
        _pass1_kernel,
        grid_spec=grid_spec,
        out_shape=[
            jax.ShapeDtypeStruct((nb, 1, _BR), jnp.float32),
            jax.ShapeDtypeStruct((nb, 1, _BR), jnp.float32),
        ],
        compiler_params=pltpu.CompilerParams(
            dimension_semantics=("parallel",)
        ),
    )(y_true, cosine, *([cosine] * _BR))
    sums = sums3.reshape(B)
    tgt = tgt3.reshape(B)

    total = jnp.sum(sums)
    b_batch = (total - jnp.sum(jnp.exp(tgt * _PREV_S))) / B
    med_cos = jnp.median(tgt)
    running_b = _RUNNING_B * _MOMENTUM + b_batch * (1.0 - _MOMENTUM)
    running_cos = _RUNNING_COS * _MOMENTUM + med_cos * (1.0 - _MOMENTUM)
    prev_s = jnp.log(running_b) / (jnp.maximum(running_cos, 0.7) - _MARGIN)
    prev_s = jnp.minimum(prev_s, _MAX_S)

    def _fast(_):
        return jnp.mean(jnp.log(sums) - prev_s * tgt)

    def _slow(_):
        sums2 = pl.pallas_call(
            _pass2_kernel,
            grid=(nb,),
            in_specs=[
                pl.BlockSpec(memory_space=pltpu.SMEM),
                pl.BlockSpec((_BR, C), lambda i: (i, 0)),
            ],
            out_specs=pl.BlockSpec((1, 1, _BR), lambda i: (i, 0, 0)),
            out_shape=jax.ShapeDtypeStruct((nb, 1, _BR), jnp.float32),
        )(prev_s[None], cosine)
        return jnp.mean(jnp.log(sums2.reshape(B)) - prev_s * tgt)

    return jax.lax.cond(prev_s == _PREV_S, _fast, _slow, None)


# speedup vs baseline: 1.1379x; 1.0516x over previous
import jax
import jax.numpy as jnp
from jax.experimental import pallas as pl
from jax.experimental.pallas import tpu as pltpu

_MARGIN = 0.0
_MOMENTUM = 0.95
_MAX_S = 20.0
_PREV_S = 20.0
_RUNNING_B = 1000.0
_RUNNING_COS = 0.7

_BR = 32
_LANES = 128


def _pass1_kernel(y_sm, x_ref, *args):
    seg_refs = args[:_BR]
    sums_ref, tgt_ref = args[_BR], args[_BR + 1]
    i = pl.program_id(0)
    x = x_ref[...]
    sums_ref[...] = jnp.sum(jnp.exp(x * _PREV_S), axis=1).reshape(1, 1, _BR)
    parts = []
    for r in range(_BR):
        lane = y_sm[i * _BR + r] % _LANES
        m = jax.lax.broadcasted_iota(jnp.int32, (1, _LANES), 1) == lane
        parts.append(jnp.where(m, seg_refs[r][pl.ds(r, 1), :], 0.0))
    tgt = jnp.sum(jnp.concatenate(parts, axis=0), axis=1)
    tgt_ref[...] = tgt.reshape(1, 1, _BR)


def _pass2_kernel(s_ref, x_ref, sums_ref):
    x = x_ref[...]
    sums_ref[...] = jnp.sum(jnp.exp(x * s_ref[0]), axis=1).reshape(1, 1, _BR)


def kernel(cosine, y_true):
    B, C = cosine.shape
    y_true = y_true.astype(jnp.int32)
    nb = B // _BR

    def _seg_spec(r):
        return pl.BlockSpec(
            (_BR, _LANES), lambda i, y: (i, y[i * _BR + r] // _LANES)
        )

    grid_spec = pltpu.PrefetchScalarGridSpec(
        num_scalar_prefetch=1,
        grid=(nb,),
        in_specs=[
            pl.BlockSpec((_BR, C), lambda i, y: (i, 0)),
            *[_seg_spec(r) for r in range(_BR)],
        ],
        out_specs=[
            pl.BlockSpec((1, 1, _BR), lambda i, y: (i, 0, 0)),
            pl.BlockSpec((1, 1, _BR), lambda i, y: (i, 0, 0)),
        ],
    )
    sums3, tgt3 = pl.pallas_call(
        _pass1_kernel,
        grid_spec=grid_spec,
        out_shape=[
            jax.ShapeDtypeStruct((nb, 1, _BR), jnp.float32),
            jax.ShapeDtypeStruct((nb, 1, _BR), jnp.float32),
        ],
        compiler_params=pltpu.CompilerParams(
            dimension_semantics=("parallel",)
        ),
    )(y_true, cosine, *([cosine] * _BR))
    sums = sums3.reshape(B)
    tgt = tgt3.reshape(B)

    total = jnp.sum(sums)
    b_batch = (total - jnp.sum(jnp.exp(tgt * _PREV_S))) / B
    med_cos = jnp.median(tgt)
    running_b = _RUNNING_B * _MOMENTUM + b_batch * (1.0 - _MOMENTUM)
    running_cos = _RUNNING_COS * _MOMENTUM + med_cos * (1.0 - _MOMENTUM)
    prev_s = jnp.log(running_b) / (jnp.maximum(running_cos, 0.7) - _MARGIN)
    prev_s = jnp.minimum(prev_s, _MAX_S)

    def _fast(_):
        return jnp.mean(jnp.log(sums) - prev_s * tgt)

    def _slow(_):
        sums2 = pl.pallas_call(
            _pass2_kernel,
            grid=(nb,),
            in_specs=[
                pl.BlockSpec(memory_space=pltpu.SMEM),
                pl.BlockSpec((_BR, C), lambda i: (i, 0)),
            ],
            out_specs=pl.BlockSpec((1, 1, _BR), lambda i: (i, 0, 0)),
            out_shape=jax.ShapeDtypeStruct((nb, 1, _BR), jnp.float32),
        )(prev_s[None], cosine)
        return jnp.mean(jnp.log(sums2.reshape(B)) - prev_s * tgt)

    return jax.lax.cond(prev_s == _PREV_S, _fast, _slow, None)
